# baseline (speedup 1.0000x reference)
import functools

import jax
import jax.numpy as jnp
from jax import lax
from jax.experimental import pallas as pl
from jax.experimental.pallas import tpu as pltpu
from jax.experimental.pallas import tpu_sc as plsc

NC = 2
NS = 16
NW = NC * NS
L = 16


def _make_transpose(D, V, NBUF=6):
    NU = V // 128
    TAIL = V - NU * 128
    per_w = (NU + NW - 1) // NW
    mesh = plsc.VectorSubcoreMesh(core_axis_name="c", subcore_axis_name="s")

    @functools.partial(
        pl.kernel,
        mesh=mesh,
        out_type=jax.ShapeDtypeStruct((V // 2, 2 * D), jnp.float32),
        scratch_types=[
            [pltpu.VMEM((D, 128), jnp.float32) for _ in range(NBUF)],
            [pltpu.VMEM((D, 128), jnp.float32) for _ in range(NBUF)],
            [pltpu.SemaphoreType.DMA for _ in range(NBUF)],
            [pltpu.SemaphoreType.DMA for _ in range(NBUF)],
        ],
        compiler_params=pltpu.CompilerParams(needs_layout_passes=False),
    )
    def tr(tt_hbm, tail_hbm, tp_hbm, inb, outb, r_sems, w_sems):
        wid = lax.axis_index("s") * NC + lax.axis_index("c")
        iota = lax.iota(jnp.int32, L)

        def rd_start(u, b):
            pltpu.async_copy(
                tt_hbm.at[:, pl.ds(u * 128, 128)], inb[b], r_sems[b]
            )

        def rd_wait(b):
            pltpu.make_async_copy(
                tt_hbm.at[:, pl.ds(0, 128)], inb[b], r_sems[b]
            ).wait()

        def wr_start(u, b):
            pltpu.async_copy(outb[b], tp_hbm.at[pl.ds(u * D, D)], w_sems[b])

        def wr_wait(b):
            pltpu.make_async_copy(
                outb[b], tp_hbm.at[pl.ds(0, D)], w_sems[b]
            ).wait()

        def transpose_block(b, nrows):
            for rg in range(nrows // L):
                rvec = iota + rg * L
                col2 = 2 * rvec

                @plsc.parallel_loop(0, D, 1, unroll=8)
                def _(s, _b=b, _rvec=rvec, _col2=col2):
                    kv = lax.bitwise_and(s + iota, D - 1)
                    for h in range(2):
                        vals = plsc.load_gather(inb[_b], [kv, _col2 + h])
                        plsc.store_scatter(
                            outb[_b], [_rvec, h * D + kv], vals
                        )

        for b in range(NBUF):
            rd_start(wid + b * NW, b)

        def step(i, carry):
            for b in range(NBUF):
                u = wid + (i + b) * NW

                @pl.when(u < NU)
                def _():
                    rd_wait(b)

                    @pl.when(i + b >= NBUF)
                    def _():
                        wr_wait(b)

                    transpose_block(b, D)
                    wr_start(u, b)
                    nu = wid + (i + b + NBUF) * NW

                    @pl.when(nu < NU)
                    def _():
                        rd_start(nu, b)

            return carry

        nsteps = (per_w + NBUF - 1) // NBUF
        lax.fori_loop(0, nsteps, lambda t, c: step(t * NBUF, c), 0,
                      unroll=False)

        for b in range(NBUF):
            wr_wait(b)

        @pl.when(wid == 0)
        def _():
            pltpu.sync_copy(tail_hbm, outb[0].at[pl.ds(0, TAIL // 2)])
            pltpu.sync_copy(
                outb[0].at[pl.ds(0, TAIL // 2)],
                tp_hbm.at[pl.ds(NU * D, TAIL // 2)],
            )

    return tr


def _make_lookup(R, C, D, NBUF=2):
    bw = R // NW
    ng = bw // L
    mesh = plsc.VectorSubcoreMesh(core_axis_name="c", subcore_axis_name="s")

    @functools.partial(
        pl.kernel,
        mesh=mesh,
        out_type=jax.ShapeDtypeStruct((C, D, R), jnp.float32),
        scratch_types=[
            pltpu.VMEM((C, bw), jnp.int32),
            [pltpu.VMEM((bw,), jnp.int32) for _ in range(NBUF)],
            [pltpu.VMEM((bw, 2 * D), jnp.float32) for _ in range(NBUF)],
            [pltpu.VMEM((D, bw), jnp.float32) for _ in range(NBUF)],
            [pltpu.SemaphoreType.DMA for _ in range(NBUF)],
            [pltpu.SemaphoreType.DMA for _ in range(NBUF)],
        ],
        compiler_params=pltpu.CompilerParams(needs_layout_passes=False),
    )
    def lookup(xt_hbm, tp_hbm, out_hbm, idx_v, pidx, bufs, fms, g_sems, w_sems):
        wid = lax.axis_index("s") * NC + lax.axis_index("c")
        col0 = wid * bw
        pltpu.sync_copy(xt_hbm.at[:, pl.ds(col0, bw)], idx_v)

        iota = lax.iota(jnp.int32, L)

        def gather_start(j, b):
            for g in range(ng):
                v = idx_v[j, pl.ds(g * L, L)]
                pidx[b][pl.ds(g * L, L)] = lax.shift_right_logical(v, 1)
            pltpu.async_copy(tp_hbm.at[pidx[b]], bufs[b], g_sems[b])

        def gather_wait(b):
            pltpu.make_async_copy(tp_hbm.at[pidx[b]], bufs[b], g_sems[b]).wait()

        def write_start(j, b):
            pltpu.async_copy(
                fms[b], out_hbm.at[j, :, pl.ds(col0, bw)], w_sems[b]
            )

        def write_wait(b):
            pltpu.make_async_copy(
                fms[b], out_hbm.at[0, :, pl.ds(col0, bw)], w_sems[b]
            ).wait()

        for b in range(NBUF):
            gather_start(b, b)

        def body(jj, carry):
            for b in range(NBUF):
                j = jj + b
                gather_wait(b)

                @pl.when(j >= NBUF)
                def _():
                    write_wait(b)

                for g in range(ng):
                    v = idx_v[j, pl.ds(g * L, L)]
                    row = iota + (g * L)
                    colb = lax.bitwise_and(v, 1) * D

                    @plsc.parallel_loop(0, D, 1, unroll=8)
                    def _(k, _row=row, _colb=colb, _b=b, _g=g):
                        m = lax.bitwise_and(k + iota, D - 1)
                        vals = plsc.load_gather(
                            bufs[_b], [_row, _colb + m]
                        )
                        plsc.store_scatter(fms[_b], [m, _row], vals)

                write_start(j, b)

                @pl.when(j + NBUF < C)
                def _():
                    gather_start(j + NBUF, b)

            return carry

        lax.fori_loop(0, C // NBUF, lambda t, c: body(t * NBUF, c), 0,
                      unroll=False)

        for b in range(NBUF):
            write_wait(b)

    return lookup


def kernel(x, table):
    R, C = x.shape
    V, D = table.shape
    xt = x.T.astype(jnp.int32)
    nfull = (V // 128) * 128
    tail = table[nfull:].reshape((V - nfull) // 2, 2 * D)
    tp = _make_transpose(D, V)(table.T, tail)
    out_st = _make_lookup(R, C, D)(xt, tp)
    return jnp.transpose(out_st, (2, 0, 1))


# speedup vs baseline: 1.0211x; 1.0211x over previous
import functools

import jax
import jax.numpy as jnp
from jax import lax
from jax.experimental import pallas as pl
from jax.experimental.pallas import tpu as pltpu
from jax.experimental.pallas import tpu_sc as plsc

NC = 2
NS = 16
NW = NC * NS
L = 16


def _make_transpose(D, V, NBUF=4):
    NU = V // 128
    TAIL = V - NU * 128
    per_w = (NU + NW - 1) // NW
    mesh = plsc.VectorSubcoreMesh(core_axis_name="c", subcore_axis_name="s")

    @functools.partial(
        pl.kernel,
        mesh=mesh,
        out_type=jax.ShapeDtypeStruct((V // 2, 2 * D), jnp.float32),
        scratch_types=[
            [pltpu.VMEM((D, 128), jnp.float32) for _ in range(NBUF)],
            [pltpu.VMEM((D, 128), jnp.float32) for _ in range(NBUF)],
            [pltpu.SemaphoreType.DMA for _ in range(NBUF)],
            [pltpu.SemaphoreType.DMA for _ in range(NBUF)],
        ],
        compiler_params=pltpu.CompilerParams(needs_layout_passes=False),
    )
    def tr(tt_hbm, tail_hbm, tp_hbm, inb, outb, r_sems, w_sems):
        wid = lax.axis_index("s") * NC + lax.axis_index("c")
        iota = lax.iota(jnp.int32, L)

        def rd_start(u, b):
            pltpu.async_copy(
                tt_hbm.at[:, pl.ds(u * 128, 128)], inb[b], r_sems[b]
            )

        def rd_wait(b):
            pltpu.make_async_copy(
                tt_hbm.at[:, pl.ds(0, 128)], inb[b], r_sems[b]
            ).wait()

        def wr_start(u, b):
            pltpu.async_copy(outb[b], tp_hbm.at[pl.ds(u * D, D)], w_sems[b])

        def wr_wait(b):
            pltpu.make_async_copy(
                outb[b], tp_hbm.at[pl.ds(0, D)], w_sems[b]
            ).wait()

        def transpose_block(b, nrows):
            for rg in range(nrows // L):
                rvec = iota + rg * L
                col2 = 2 * rvec

                @plsc.parallel_loop(0, D, 1, unroll=8)
                def _(s, _b=b, _rvec=rvec, _col2=col2):
                    kv = lax.bitwise_and(s + iota, D - 1)
                    for h in range(2):
                        vals = plsc.load_gather(inb[_b], [kv, _col2 + h])
                        plsc.store_scatter(
                            outb[_b], [_rvec, h * D + kv], vals
                        )

        for b in range(NBUF):
            rd_start(wid + b * NW, b)

        def step(i, carry):
            for b in range(NBUF):
                u = wid + (i + b) * NW

                @pl.when(u < NU)
                def _():
                    rd_wait(b)

                    @pl.when(i + b >= NBUF)
                    def _():
                        wr_wait(b)

                    transpose_block(b, D)
                    wr_start(u, b)
                    nu = wid + (i + b + NBUF) * NW

                    @pl.when(nu < NU)
                    def _():
                        rd_start(nu, b)

            return carry

        nsteps = (per_w + NBUF - 1) // NBUF
        lax.fori_loop(0, nsteps, lambda t, c: step(t * NBUF, c), 0,
                      unroll=False)

        for b in range(NBUF):
            wr_wait(b)

        @pl.when(wid == 0)
        def _():
            pltpu.sync_copy(tail_hbm, outb[0].at[pl.ds(0, TAIL // 2)])
            pltpu.sync_copy(
                outb[0].at[pl.ds(0, TAIL // 2)],
                tp_hbm.at[pl.ds(NU * D, TAIL // 2)],
            )

    return tr


def _make_lookup(R, C, D, NBUF=2):
    bw = R // NW
    ng = bw // L
    mesh = plsc.VectorSubcoreMesh(core_axis_name="c", subcore_axis_name="s")

    @functools.partial(
        pl.kernel,
        mesh=mesh,
        out_type=jax.ShapeDtypeStruct((C, D, R), jnp.float32),
        scratch_types=[
            pltpu.VMEM((C, bw), jnp.int32),
            [pltpu.VMEM((bw,), jnp.int32) for _ in range(NBUF)],
            [pltpu.VMEM((bw, 2 * D), jnp.float32) for _ in range(NBUF)],
            [pltpu.VMEM((D, bw), jnp.float32) for _ in range(NBUF)],
            [pltpu.SemaphoreType.DMA for _ in range(NBUF)],
            [pltpu.SemaphoreType.DMA for _ in range(NBUF)],
        ],
        compiler_params=pltpu.CompilerParams(needs_layout_passes=False),
    )
    def lookup(xt_hbm, tp_hbm, out_hbm, idx_v, pidx, bufs, fms, g_sems, w_sems):
        wid = lax.axis_index("s") * NC + lax.axis_index("c")
        col0 = wid * bw
        pltpu.sync_copy(xt_hbm.at[:, pl.ds(col0, bw)], idx_v)

        iota = lax.iota(jnp.int32, L)

        def gather_start(j, b):
            for g in range(ng):
                v = idx_v[j, pl.ds(g * L, L)]
                pidx[b][pl.ds(g * L, L)] = lax.shift_right_logical(v, 1)
            pltpu.async_copy(tp_hbm.at[pidx[b]], bufs[b], g_sems[b])

        def gather_wait(b):
            pltpu.make_async_copy(tp_hbm.at[pidx[b]], bufs[b], g_sems[b]).wait()

        def write_start(j, b):
            pltpu.async_copy(
                fms[b], out_hbm.at[j, :, pl.ds(col0, bw)], w_sems[b]
            )

        def write_wait(b):
            pltpu.make_async_copy(
                fms[b], out_hbm.at[0, :, pl.ds(col0, bw)], w_sems[b]
            ).wait()

        for b in range(NBUF):
            gather_start(b, b)

        def body(jj, carry):
            for b in range(NBUF):
                j = jj + b
                gather_wait(b)

                @pl.when(j >= NBUF)
                def _():
                    write_wait(b)

                for g in range(ng):
                    v = idx_v[j, pl.ds(g * L, L)]
                    row = iota + (g * L)
                    colb = lax.bitwise_and(v, 1) * D

                    @plsc.parallel_loop(0, D, 1, unroll=8)
                    def _(k, _row=row, _colb=colb, _b=b, _g=g):
                        m = lax.bitwise_and(k + iota, D - 1)
                        vals = plsc.load_gather(
                            bufs[_b], [_row, _colb + m]
                        )
                        plsc.store_scatter(fms[_b], [m, _row], vals)

                write_start(j, b)

                @pl.when(j + NBUF < C)
                def _():
                    gather_start(j + NBUF, b)

            return carry

        lax.fori_loop(0, C // NBUF, lambda t, c: body(t * NBUF, c), 0,
                      unroll=False)

        for b in range(NBUF):
            write_wait(b)

    return lookup


def kernel(x, table):
    R, C = x.shape
    V, D = table.shape
    xt = x.T.astype(jnp.int32)
    nfull = (V // 128) * 128
    tail = table[nfull:].reshape((V - nfull) // 2, 2 * D)
    tp = _make_transpose(D, V)(table.T, tail)
    out_st = _make_lookup(R, C, D)(xt, tp)
    return jnp.transpose(out_st, (2, 0, 1))
